# Initial kernel scaffold; baseline (speedup 1.0000x reference)
import jax
import jax.numpy as jnp
from jax.experimental import pallas as pl


def kernel(x, edge_index, batch, xo, W_c1, b_c1, W_c2, b_c2, W_c3, b_c3, W_g1, b_g1, W_g2, b_g2, cw1, cb1, cw2, cb2, cw3, cb3, W_xt, b_xt, W_f1, b_f1, W_f2, b_f2, W_o, b_o):
    raise NotImplementedError("write your pallas kernel here")


# speedup vs baseline: 1.3453x; 1.3453x over previous
import functools

import jax
import jax.numpy as jnp
from jax.experimental import pallas as pl
from jax.experimental.pallas import tpu as pltpu

N = 50000
E = 800000
B = 128
L = 720

N_PAD = 50176


def _mm_body(x_ref, w_ref, bin_ref, bout_ref, o_ref, *, relu_in, relu_out):
    xt = x_ref[...]
    if bin_ref is not None:
        xt = xt + bin_ref[...]
    if relu_in:
        xt = jnp.maximum(xt, 0.0)
    acc = jnp.dot(xt, w_ref[...], preferred_element_type=jnp.float32)
    if bout_ref is not None:
        acc = acc + bout_ref[...]
    if relu_out:
        acc = jnp.maximum(acc, 0.0)
    o_ref[...] = acc


def _mm(x, W, b_in=None, b_out=None, relu_in=False, relu_out=False, tm=512):
    M, K = x.shape
    K2, Nw = W.shape
    assert K == K2 and M % tm == 0, (x.shape, W.shape, tm)
    grid = (M // tm,)
    in_specs = [pl.BlockSpec((tm, K), lambda i: (i, 0)),
                pl.BlockSpec((K, Nw), lambda i: (0, 0))]
    args = [x, W]
    if b_in is not None:
        in_specs.append(pl.BlockSpec((1, K), lambda i: (0, 0)))
        args.append(b_in.reshape(1, K))
    if b_out is not None:
        in_specs.append(pl.BlockSpec((1, Nw), lambda i: (0, 0)))
        args.append(b_out.reshape(1, Nw))

    def body(*refs):
        x_ref, w_ref = refs[0], refs[1]
        idx = 2
        bin_ref = bout_ref = None
        if b_in is not None:
            bin_ref = refs[idx]; idx += 1
        if b_out is not None:
            bout_ref = refs[idx]; idx += 1
        o_ref = refs[idx]
        _mm_body(x_ref, w_ref, bin_ref, bout_ref, o_ref,
                 relu_in=relu_in, relu_out=relu_out)

    return pl.pallas_call(
        body,
        grid=grid,
        in_specs=in_specs,
        out_specs=pl.BlockSpec((tm, Nw), lambda i: (i, 0)),
        out_shape=jax.ShapeDtypeStruct((M, Nw), jnp.float32),
    )(*args)


def _head_body(g_ref, ct_ref, wg1_ref, bg1_ref, wg2_ref, bg2_ref,
               wxt_ref, bxt_ref, wf1_ref, bf1_ref, wf2_ref, bf2_ref,
               wo_ref, bo_ref, o_ref):
    g = g_ref[...]
    g = jnp.maximum(jnp.dot(g, wg1_ref[...], preferred_element_type=jnp.float32)
                    + bg1_ref[...], 0.0)
    g = jnp.dot(g, wg2_ref[...], preferred_element_type=jnp.float32) + bg2_ref[...]
    xt = jnp.dot(ct_ref[...], wxt_ref[...], preferred_element_type=jnp.float32) \
        + bxt_ref[...]
    xc = jnp.concatenate([g, xt], axis=1)
    xc = jnp.maximum(jnp.dot(xc, wf1_ref[...], preferred_element_type=jnp.float32)
                     + bf1_ref[...], 0.0)
    xc = jnp.maximum(jnp.dot(xc, wf2_ref[...], preferred_element_type=jnp.float32)
                     + bf2_ref[...], 0.0)
    o_ref[...] = jnp.dot(xc, wo_ref[...], preferred_element_type=jnp.float32) \
        + bo_ref[...]


def _head(g, ct_flat, W_g1, b_g1, W_g2, b_g2, W_xt, b_xt,
          W_f1, b_f1, W_f2, b_f2, W_o, b_o):
    args = [g, ct_flat, W_g1, b_g1.reshape(1, -1), W_g2, b_g2.reshape(1, -1),
            W_xt, b_xt.reshape(1, -1), W_f1, b_f1.reshape(1, -1),
            W_f2, b_f2.reshape(1, -1), W_o, b_o.reshape(1, -1)]
    return pl.pallas_call(
        _head_body,
        out_shape=jax.ShapeDtypeStruct((B, 1), jnp.float32),
    )(*args)


def kernel(x, edge_index, batch, xo, W_c1, b_c1, W_c2, b_c2, W_c3, b_c3,
           W_g1, b_g1, W_g2, b_g2, cw1, cb1, cw2, cb2, cw3, cb3,
           W_xt, b_xt, W_f1, b_f1, W_f2, b_f2, W_o, b_o):
    src = edge_index[0]
    dst = edge_index[1]

    deg = jnp.zeros((N,), jnp.float32).at[dst].add(1.0) + 1.0
    dinv = jax.lax.rsqrt(deg)
    norm = dinv[src] * dinv[dst]
    dinv2 = (dinv * dinv)

    W1p = jnp.pad(W_c1, ((0, 2), (0, 2)))
    b1p = jnp.pad(b_c1, (0, 2))
    W2p = jnp.pad(W_c2, ((0, 2), (0, 4)))
    b2p = jnp.pad(b_c2, (0, 4))
    W3p = jnp.pad(W_c3, ((0, 4), (0, 8)))
    b3p = jnp.pad(b_c3, (0, 8))
    x_p = jnp.pad(x, ((0, N_PAD - N), (0, 2)))

    dinv2_col = jnp.pad(dinv2, (0, N_PAD - N))[:, None]

    def gcn_layer(h_pre, Wp, rest_relu_bias):
        h = _mm(h_pre, Wp, **rest_relu_bias)
        msg = h[:N][src] * norm[:, None]
        agg = jnp.zeros((N_PAD, h.shape[1]), jnp.float32).at[dst].add(msg)
        agg = agg + dinv2_col * h
        return agg

    agg1 = gcn_layer(x_p, W1p, {})
    agg2 = gcn_layer(agg1, W2p, dict(b_in=b1p, relu_in=True))
    agg3 = gcn_layer(agg2, W3p, dict(b_in=b2p, relu_in=True))
    h3 = jnp.maximum(agg3[:N] + b3p[None, :], 0.0)

    g = jax.ops.segment_max(h3, batch, num_segments=B)[:, :312]

    def im2col(v, k):
        T = v.shape[1]
        cols = [v[:, i:T - k + 1 + i, :] for i in range(k)]
        return jnp.concatenate(cols, axis=2)

    def pool3(v):
        T3 = (v.shape[1] // 3) * 3
        return jnp.max(v[:, :T3].reshape(B, T3 // 3, 3, v.shape[2]), axis=2)

    v = xo.transpose(0, 2, 1)
    w1 = cw1.transpose(2, 1, 0).reshape(8, 32)
    c1 = im2col(v, 8).reshape(B * 713, 8)
    c1 = jnp.pad(c1, ((0, 91392 - B * 713), (0, 0)))
    y1 = _mm(c1, w1, b_out=cb1, relu_out=True, tm=448)[:B * 713]
    v = pool3(y1.reshape(B, 713, 32))

    w2 = cw2.transpose(2, 1, 0).reshape(8 * 32, 64)
    c2 = im2col(v, 8).reshape(B * 230, 256)
    c2 = jnp.pad(c2, ((0, 29696 - B * 230), (0, 0)))
    y2 = _mm(c2, w2, b_out=cb2, relu_out=True, tm=464)[:B * 230]
    v = pool3(y2.reshape(B, 230, 64))

    w3 = cw3.transpose(2, 1, 0).reshape(8 * 64, 128)
    c3 = im2col(v, 8).reshape(B * 69, 512)
    c3 = jnp.pad(c3, ((0, 8832 - B * 69), (0, 0)))
    y3 = _mm(c3, w3, b_out=cb3, relu_out=True, tm=552)[:B * 69]
    v = pool3(y3.reshape(B, 69, 128))

    ct_flat = v.reshape(B, 23 * 128)
    W_xt_nhc = W_xt.reshape(128, 23, 128).transpose(1, 0, 2).reshape(2944, 128)

    return _head(g, ct_flat, W_g1, b_g1, W_g2, b_g2, W_xt_nhc, b_xt,
                 W_f1, b_f1, W_f2, b_f2, W_o, b_o)


# speedup vs baseline: 2.7871x; 2.0717x over previous
import functools

import jax
import jax.numpy as jnp
from jax import lax
from jax.experimental import pallas as pl
from jax.experimental.pallas import tpu as pltpu
from jax.experimental.pallas import tpu_sc as plsc

N = 50000
E = 800000
B = 128
L = 720

N_PAD = 51200
EP = 819200
EROWS = EP // 128
TROWS = EROWS // 32
DC = 16
GB = 128
ZR = 200

_MESH = dict(core_axis_name="c", subcore_axis_name="s")


def _sc_degree(dst_p):
    per_tile = EP // 32
    nb = per_tile // GB
    zchunk = N_PAD // 16

    @functools.partial(
        pl.kernel,
        out_type=jax.ShapeDtypeStruct((2 * (N_PAD + 16),), jnp.float32),
        mesh=plsc.VectorSubcoreMesh(**_MESH),
        scratch_types=[
            pltpu.VMEM((1, GB), jnp.int32),
            pltpu.VMEM((GB,), jnp.float32),
            pltpu.VMEM((8 * 16,), jnp.float32),
            pltpu.VMEM((zchunk,), jnp.float32),
            pltpu.VMEM_SHARED((N_PAD + 16,), jnp.float32),
        ],
    )
    def deg_kernel(dst_hbm, out_hbm, idxb, ones, zb, vbuf, acc):
        c = lax.axis_index("c")
        s = lax.axis_index("s")
        onev = jnp.ones((16,), jnp.float32)
        zerov = jnp.zeros((16,), jnp.float32)
        for t in range(8):
            ones[pl.ds(t * 16, 16)] = onev
            zb[pl.ds(t * 16, 16)] = zerov

        def zero_body(z, _):
            pltpu.sync_copy(zb, acc.at[pl.ds(s * zchunk + z * 128, 128)])
            return 0
        lax.fori_loop(0, zchunk // 128, zero_body, 0, unroll=False)
        plsc.subcore_barrier()

        base = (c * 16 + s) * per_tile

        def batch_body(j, _):
            pltpu.sync_copy(dst_hbm.at[pl.ds(base + j * GB, GB)], idxb.at[0])
            pltpu.sync_copy(ones, acc.at[idxb.at[0]], add=True)
            return 0
        lax.fori_loop(0, nb, batch_body, 0, unroll=False)
        plsc.subcore_barrier()

        pltpu.sync_copy(acc.at[pl.ds(s * zchunk, zchunk)], vbuf)
        pltpu.sync_copy(vbuf,
                        out_hbm.at[pl.ds(c * (N_PAD + 16) + s * zchunk,
                                         zchunk)])

    return deg_kernel(dst_p)


def _sc_gcn_scatter(h2d, srck, dstr, K3):
    zpt = N_PAD // 16 // ZR

    @functools.partial(
        pl.kernel,
        out_type=jax.ShapeDtypeStruct((2 * K3 * N_PAD, DC), jnp.float32),
        mesh=plsc.VectorSubcoreMesh(**_MESH),
        compiler_params=pltpu.CompilerParams(use_tc_tiling_on_sc=False),
        scratch_types=[
            pltpu.VMEM((TROWS, 128), jnp.int32),
            pltpu.VMEM((TROWS, 128), jnp.int32),
            pltpu.VMEM((GB, DC), jnp.float32),
            pltpu.VMEM((ZR, DC), jnp.float32),
            pltpu.VMEM((ZR, DC), jnp.float32),
            pltpu.VMEM_SHARED((N_PAD + 16, DC), jnp.float32),
            pltpu.SemaphoreType.DMA,
        ],
    )
    def scatter_kernel(h_hbm, srck_hbm, dstr_hbm, out_hbm,
                       idxs, idxd, rowbuf, zbuf, bounce, acc, sem):
        c = lax.axis_index("c")
        s = lax.axis_index("s")
        wid = c * 16 + s
        zerovf = jnp.zeros((16,), jnp.float32)
        for r in range(ZR):
            for q in range(DC // 16):
                zbuf[r, pl.ds(q * 16, 16)] = zerovf

        pltpu.sync_copy(dstr_hbm.at[pl.ds(wid * TROWS, TROWS)], idxd)

        for k in range(K3):
            pltpu.sync_copy(
                srck_hbm.at[k].at[pl.ds(wid * TROWS, TROWS)], idxs)

            def zero_body(z, _):
                pltpu.sync_copy(
                    zbuf, acc.at[pl.ds((s * zpt + z) * ZR, ZR)])
                return 0
            lax.fori_loop(0, zpt, zero_body, 0, unroll=False)
            plsc.subcore_barrier()

            def batch_body(j, _):
                pltpu.async_copy(h_hbm.at[idxs.at[j]], rowbuf, sem).wait()
                pltpu.sync_copy(rowbuf, acc.at[idxd.at[j]], add=True)
                return 0
            lax.fori_loop(0, TROWS, batch_body, 0, unroll=False)
            plsc.subcore_barrier()

            obase = (c * K3 + k) * N_PAD

            def wout_body(z, _):
                pltpu.sync_copy(acc.at[pl.ds((s * zpt + z) * ZR, ZR)],
                                bounce)
                pltpu.sync_copy(
                    bounce,
                    out_hbm.at[pl.ds(obase + (s * zpt + z) * ZR, ZR)])
                return 0
            lax.fori_loop(0, zpt, wout_body, 0, unroll=False)
            plsc.subcore_barrier()

    return scatter_kernel(h2d, srck, dstr)


def _to_cm(h, K3, tm):
    return h.reshape(tm, K3, DC).transpose(1, 0, 2)


def _from_cm(h_cm):
    K3, tm, _ = h_cm.shape
    return h_cm.transpose(1, 0, 2).reshape(tm, K3 * DC)


def _mm1_scaled(x_p, W1p, deg_parts, K3, tm=512):
    M, K = x_p.shape
    _, Nw = W1p.shape

    def body(x_ref, w_ref, dg_ref, o_ref, dinv_ref):
        dg = dg_ref[0, :] + dg_ref[1, :] + 1.0
        dinv = lax.rsqrt(dg)[:, None]
        h = jnp.dot(x_ref[...], w_ref[...],
                    preferred_element_type=jnp.float32) * dinv
        o_ref[...] = _to_cm(h, K3, tm)
        dinv_ref[...] = dinv

    return pl.pallas_call(
        body, grid=(M // tm,),
        in_specs=[pl.BlockSpec((tm, K), lambda i: (i, 0)),
                  pl.BlockSpec((K, Nw), lambda i: (0, 0)),
                  pl.BlockSpec((2, tm), lambda i: (0, i))],
        out_specs=[pl.BlockSpec((K3, tm, DC), lambda i: (0, i, 0)),
                   pl.BlockSpec((tm, 1), lambda i: (i, 0))],
        out_shape=[jax.ShapeDtypeStruct((K3, M, DC), jnp.float32),
                   jax.ShapeDtypeStruct((M, 1), jnp.float32)],
    )(x_p, W1p, deg_parts)


def _mm_layer(parts, h_cm, dinv, b_in, W, K3o, tm=512):
    _, K3i, M, _ = parts.shape
    K = K3i * DC
    _, Nw = W.shape

    def body(p0_ref, p1_ref, h_ref, d_ref, b_ref, w_ref, o_ref):
        agg = p0_ref[0] + p1_ref[0]
        dinv_t = d_ref[...]
        pre = jnp.maximum(
            (_from_cm(agg) + _from_cm(h_ref[...])) * dinv_t + b_ref[...],
            0.0)
        h2 = jnp.dot(pre, w_ref[...],
                     preferred_element_type=jnp.float32) * dinv_t
        o_ref[...] = _to_cm(h2, K3o, tm)

    return pl.pallas_call(
        body, grid=(M // tm,),
        in_specs=[pl.BlockSpec((1, K3i, tm, DC), lambda i: (0, 0, i, 0)),
                  pl.BlockSpec((1, K3i, tm, DC), lambda i: (1, 0, i, 0)),
                  pl.BlockSpec((K3i, tm, DC), lambda i: (0, i, 0)),
                  pl.BlockSpec((tm, 1), lambda i: (i, 0)),
                  pl.BlockSpec((1, K), lambda i: (0, 0)),
                  pl.BlockSpec((K, Nw), lambda i: (0, 0))],
        out_specs=pl.BlockSpec((K3o, tm, DC), lambda i: (0, i, 0)),
        out_shape=jax.ShapeDtypeStruct((K3o, M, DC), jnp.float32),
    )(parts, parts, h_cm, dinv, b_in.reshape(1, K), W)


def _post3(parts, h_cm, dinv, b_in, tm=512):
    _, K3i, M, _ = parts.shape
    K = K3i * DC

    def body(p0_ref, p1_ref, h_ref, d_ref, b_ref, o_ref):
        agg = p0_ref[0] + p1_ref[0]
        o_ref[...] = jnp.maximum(
            (_from_cm(agg) + _from_cm(h_ref[...])) * d_ref[...]
            + b_ref[...], 0.0)

    return pl.pallas_call(
        body, grid=(M // tm,),
        in_specs=[pl.BlockSpec((1, K3i, tm, DC), lambda i: (0, 0, i, 0)),
                  pl.BlockSpec((1, K3i, tm, DC), lambda i: (1, 0, i, 0)),
                  pl.BlockSpec((K3i, tm, DC), lambda i: (0, i, 0)),
                  pl.BlockSpec((tm, 1), lambda i: (i, 0)),
                  pl.BlockSpec((1, K), lambda i: (0, 0))],
        out_specs=pl.BlockSpec((tm, K), lambda i: (i, 0)),
        out_shape=jax.ShapeDtypeStruct((M, K), jnp.float32),
    )(parts, parts, h_cm, dinv, b_in.reshape(1, K))


def _mm(x, W, b_out=None, relu_out=False, tm=512):
    M, K = x.shape
    _, Nw = W.shape
    in_specs = [pl.BlockSpec((tm, K), lambda i: (i, 0)),
                pl.BlockSpec((K, Nw), lambda i: (0, 0))]
    args = [x, W]
    if b_out is not None:
        in_specs.append(pl.BlockSpec((1, Nw), lambda i: (0, 0)))
        args.append(b_out.reshape(1, Nw))

    def body(*refs):
        acc = jnp.dot(refs[0][...], refs[1][...],
                      preferred_element_type=jnp.float32)
        idx = 2
        if b_out is not None:
            acc = acc + refs[idx][...]; idx += 1
        if relu_out:
            acc = jnp.maximum(acc, 0.0)
        refs[idx][...] = acc

    return pl.pallas_call(
        body, grid=(M // tm,),
        in_specs=in_specs,
        out_specs=pl.BlockSpec((tm, Nw), lambda i: (i, 0)),
        out_shape=jax.ShapeDtypeStruct((M, Nw), jnp.float32),
    )(*args)


def _head_body(g_ref, ct_ref, wg1_ref, bg1_ref, wg2_ref, bg2_ref,
               wxt_ref, bxt_ref, wf1_ref, bf1_ref, wf2_ref, bf2_ref,
               wo_ref, bo_ref, o_ref):
    g = g_ref[...]
    g = jnp.maximum(jnp.dot(g, wg1_ref[...], preferred_element_type=jnp.float32)
                    + bg1_ref[...], 0.0)
    g = jnp.dot(g, wg2_ref[...], preferred_element_type=jnp.float32) + bg2_ref[...]
    xt = jnp.dot(ct_ref[...], wxt_ref[...], preferred_element_type=jnp.float32) \
        + bxt_ref[...]
    xc = jnp.concatenate([g, xt], axis=1)
    xc = jnp.maximum(jnp.dot(xc, wf1_ref[...], preferred_element_type=jnp.float32)
                     + bf1_ref[...], 0.0)
    xc = jnp.maximum(jnp.dot(xc, wf2_ref[...], preferred_element_type=jnp.float32)
                     + bf2_ref[...], 0.0)
    o_ref[...] = jnp.dot(xc, wo_ref[...], preferred_element_type=jnp.float32) \
        + bo_ref[...]


def _head(g, ct_flat, W_g1, b_g1, W_g2, b_g2, W_xt, b_xt,
          W_f1, b_f1, W_f2, b_f2, W_o, b_o):
    args = [g, ct_flat, W_g1, b_g1.reshape(1, -1), W_g2, b_g2.reshape(1, -1),
            W_xt, b_xt.reshape(1, -1), W_f1, b_f1.reshape(1, -1),
            W_f2, b_f2.reshape(1, -1), W_o, b_o.reshape(1, -1)]
    return pl.pallas_call(
        _head_body,
        out_shape=jax.ShapeDtypeStruct((B, 1), jnp.float32),
    )(*args)


def kernel(x, edge_index, batch, xo, W_c1, b_c1, W_c2, b_c2, W_c3, b_c3,
           W_g1, b_g1, W_g2, b_g2, cw1, cb1, cw2, cb2, cw3, cb3,
           W_xt, b_xt, W_f1, b_f1, W_f2, b_f2, W_o, b_o):
    src_p = jnp.pad(edge_index[0], (0, EP - E))
    dst_p = jnp.pad(edge_index[1], (0, EP - E), constant_values=N_PAD)
    dstr = dst_p.reshape(EROWS, 128)

    def srck(K3):
        off = (jnp.arange(K3, dtype=jnp.int32) * N_PAD)[:, None]
        return (src_p[None, :] + off).reshape(K3, EROWS, 128)

    deg_parts = _sc_degree(dst_p).reshape(2, N_PAD + 16)[:, :N_PAD]

    W1p = jnp.pad(W_c1, ((0, 2), (0, 2)))
    b1p = jnp.pad(b_c1, (0, 2))
    W2p = jnp.pad(W_c2, ((0, 2), (0, 4)))
    b2p = jnp.pad(b_c2, (0, 4))
    W3p = jnp.pad(W_c3, ((0, 4), (0, 8)))
    b3p = jnp.pad(b_c3, (0, 8))
    x_p = jnp.pad(x, ((0, N_PAD - N), (0, 2)))

    h1, dinv = _mm1_scaled(x_p, W1p, deg_parts, 5)
    agg1 = _sc_gcn_scatter(h1.reshape(5 * N_PAD, DC), srck(5), dstr, 5)
    h2 = _mm_layer(agg1.reshape(2, 5, N_PAD, DC), h1, dinv, b1p, W2p, 10)
    agg2 = _sc_gcn_scatter(h2.reshape(10 * N_PAD, DC), srck(10), dstr, 10)
    h3 = _mm_layer(agg2.reshape(2, 10, N_PAD, DC), h2, dinv, b2p, W3p, 20)
    agg3 = _sc_gcn_scatter(h3.reshape(20 * N_PAD, DC), srck(20), dstr, 20)
    h3f = _post3(agg3.reshape(2, 20, N_PAD, DC), h3, dinv, b3p)

    g = jax.ops.segment_max(h3f[:N, :312], batch, num_segments=B)

    def im2col(v, k):
        T = v.shape[1]
        cols = [v[:, i:T - k + 1 + i, :] for i in range(k)]
        return jnp.concatenate(cols, axis=2)

    def pool3(v):
        T3 = (v.shape[1] // 3) * 3
        return jnp.max(v[:, :T3].reshape(B, T3 // 3, 3, v.shape[2]), axis=2)

    v = xo.transpose(0, 2, 1)
    w1 = cw1.transpose(2, 1, 0).reshape(8, 32)
    c1 = im2col(v, 8).reshape(B * 713, 8)
    c1 = jnp.pad(c1, ((0, 91392 - B * 713), (0, 0)))
    y1 = _mm(c1, w1, b_out=cb1, relu_out=True, tm=448)[:B * 713]
    v = pool3(y1.reshape(B, 713, 32))

    w2 = cw2.transpose(2, 1, 0).reshape(8 * 32, 64)
    c2 = im2col(v, 8).reshape(B * 230, 256)
    c2 = jnp.pad(c2, ((0, 29696 - B * 230), (0, 0)))
    y2 = _mm(c2, w2, b_out=cb2, relu_out=True, tm=464)[:B * 230]
    v = pool3(y2.reshape(B, 230, 64))

    w3 = cw3.transpose(2, 1, 0).reshape(8 * 64, 128)
    c3 = im2col(v, 8).reshape(B * 69, 512)
    y3 = _mm(c3, w3, b_out=cb3, relu_out=True, tm=552)[:B * 69]
    v = pool3(y3.reshape(B, 69, 128))

    ct_flat = v.reshape(B, 23 * 128)
    W_xt_nhc = W_xt.reshape(128, 23, 128).transpose(1, 0, 2).reshape(2944, 128)

    return _head(g, ct_flat, W_g1, b_g1, W_g2, b_g2, W_xt_nhc, b_xt,
                 W_f1, b_f1, W_f2, b_f2, W_o, b_o)


# speedup vs baseline: 3.0002x; 1.0765x over previous
import functools

import jax
import jax.numpy as jnp
from jax import lax
from jax.experimental import pallas as pl
from jax.experimental.pallas import tpu as pltpu
from jax.experimental.pallas import tpu_sc as plsc

N = 50000
E = 800000
B = 128
L = 720

N_PAD = 51200
EP = 819200
EROWS = EP // 128
TROWS = EROWS // 32
DC = 16
GB = 128
ZR = 200

_MESH = dict(core_axis_name="c", subcore_axis_name="s")


def _sc_degree(dst_p):
    per_tile = EP // 32
    nb = per_tile // GB
    zchunk = N_PAD // 16

    @functools.partial(
        pl.kernel,
        out_type=jax.ShapeDtypeStruct((2 * (N_PAD + 16),), jnp.float32),
        mesh=plsc.VectorSubcoreMesh(**_MESH),
        scratch_types=[
            pltpu.VMEM((1, GB), jnp.int32),
            pltpu.VMEM((GB,), jnp.float32),
            pltpu.VMEM((8 * 16,), jnp.float32),
            pltpu.VMEM((zchunk,), jnp.float32),
            pltpu.VMEM_SHARED((N_PAD + 16,), jnp.float32),
        ],
    )
    def deg_kernel(dst_hbm, out_hbm, idxb, ones, zb, vbuf, acc):
        c = lax.axis_index("c")
        s = lax.axis_index("s")
        onev = jnp.ones((16,), jnp.float32)
        zerov = jnp.zeros((16,), jnp.float32)
        for t in range(8):
            ones[pl.ds(t * 16, 16)] = onev
            zb[pl.ds(t * 16, 16)] = zerov

        def zero_body(z, _):
            pltpu.sync_copy(zb, acc.at[pl.ds(s * zchunk + z * 128, 128)])
            return 0
        lax.fori_loop(0, zchunk // 128, zero_body, 0, unroll=False)
        plsc.subcore_barrier()

        base = (c * 16 + s) * per_tile

        def batch_body(j, _):
            pltpu.sync_copy(dst_hbm.at[pl.ds(base + j * GB, GB)], idxb.at[0])
            pltpu.sync_copy(ones, acc.at[idxb.at[0]], add=True)
            return 0
        lax.fori_loop(0, nb, batch_body, 0, unroll=False)
        plsc.subcore_barrier()

        pltpu.sync_copy(acc.at[pl.ds(s * zchunk, zchunk)], vbuf)
        pltpu.sync_copy(vbuf,
                        out_hbm.at[pl.ds(c * (N_PAD + 16) + s * zchunk,
                                         zchunk)])

    return deg_kernel(dst_p)


def _sc_gcn_scatter(h2d, srck, dstr, K3):
    zpt = N_PAD // 16 // ZR

    @functools.partial(
        pl.kernel,
        out_type=jax.ShapeDtypeStruct((2 * K3 * N_PAD, DC), jnp.float32),
        mesh=plsc.VectorSubcoreMesh(**_MESH),
        compiler_params=pltpu.CompilerParams(use_tc_tiling_on_sc=False),
        scratch_types=[
            pltpu.VMEM((TROWS, 128), jnp.int32),
            pltpu.VMEM((TROWS, 128), jnp.int32),
            pltpu.VMEM((GB, DC), jnp.float32),
            pltpu.VMEM((GB, DC), jnp.float32),
            pltpu.VMEM((ZR, DC), jnp.float32),
            pltpu.VMEM((ZR, DC), jnp.float32),
            pltpu.VMEM_SHARED((N_PAD + 16, DC), jnp.float32),
            pltpu.SemaphoreType.DMA,
            pltpu.SemaphoreType.DMA,
        ],
    )
    def scatter_kernel(h_hbm, srck_hbm, dstr_hbm, out_hbm,
                       idxs, idxd, rb0, rb1, zbuf, bounce, acc, sem0, sem1):
        c = lax.axis_index("c")
        s = lax.axis_index("s")
        wid = c * 16 + s
        zerovf = jnp.zeros((16,), jnp.float32)
        for r in range(ZR):
            for q in range(DC // 16):
                zbuf[r, pl.ds(q * 16, 16)] = zerovf

        pltpu.sync_copy(dstr_hbm.at[pl.ds(wid * TROWS, TROWS)], idxd)

        for k in range(K3):
            pltpu.sync_copy(
                srck_hbm.at[k].at[pl.ds(wid * TROWS, TROWS)], idxs)

            def zero_body(z, _):
                pltpu.sync_copy(
                    zbuf, acc.at[pl.ds((s * zpt + z) * ZR, ZR)])
                return 0
            lax.fori_loop(0, zpt, zero_body, 0, unroll=False)
            plsc.subcore_barrier()

            pltpu.async_copy(h_hbm.at[idxs.at[0]], rb0, sem0)

            def pair_body(t, _):
                jj = 2 * t
                pltpu.make_async_copy(h_hbm.at[idxs.at[jj]], rb0,
                                      sem0).wait()
                pltpu.async_copy(h_hbm.at[idxs.at[jj + 1]], rb1, sem1)
                pltpu.sync_copy(rb0, acc.at[idxd.at[jj]], add=True)
                pltpu.make_async_copy(h_hbm.at[idxs.at[jj + 1]], rb1,
                                      sem1).wait()

                @pl.when(t < TROWS // 2 - 1)
                def _():
                    pltpu.async_copy(h_hbm.at[idxs.at[jj + 2]], rb0, sem0)
                pltpu.sync_copy(rb1, acc.at[idxd.at[jj + 1]], add=True)
                return 0
            lax.fori_loop(0, TROWS // 2, pair_body, 0, unroll=False)
            plsc.subcore_barrier()

            obase = (c * K3 + k) * N_PAD

            def wout_body(z, _):
                pltpu.sync_copy(acc.at[pl.ds((s * zpt + z) * ZR, ZR)],
                                bounce)
                pltpu.sync_copy(
                    bounce,
                    out_hbm.at[pl.ds(obase + (s * zpt + z) * ZR, ZR)])
                return 0
            lax.fori_loop(0, zpt, wout_body, 0, unroll=False)
            plsc.subcore_barrier()

    return scatter_kernel(h2d, srck, dstr)


def _to_cm(h, K3, tm):
    return h.reshape(tm, K3, DC).transpose(1, 0, 2)


def _from_cm(h_cm):
    K3, tm, _ = h_cm.shape
    return h_cm.transpose(1, 0, 2).reshape(tm, K3 * DC)


def _mm1_scaled(x_p, W1p, deg_parts, K3, tm=512):
    M, K = x_p.shape
    _, Nw = W1p.shape

    def body(x_ref, w_ref, dg_ref, o_ref, dinv_ref):
        dg = dg_ref[0, :] + dg_ref[1, :] + 1.0
        dinv = lax.rsqrt(dg)[:, None]
        h = jnp.dot(x_ref[...], w_ref[...],
                    preferred_element_type=jnp.float32) * dinv
        o_ref[...] = _to_cm(h, K3, tm)
        dinv_ref[...] = dinv

    return pl.pallas_call(
        body, grid=(M // tm,),
        in_specs=[pl.BlockSpec((tm, K), lambda i: (i, 0)),
                  pl.BlockSpec((K, Nw), lambda i: (0, 0)),
                  pl.BlockSpec((2, tm), lambda i: (0, i))],
        out_specs=[pl.BlockSpec((K3, tm, DC), lambda i: (0, i, 0)),
                   pl.BlockSpec((tm, 1), lambda i: (i, 0))],
        out_shape=[jax.ShapeDtypeStruct((K3, M, DC), jnp.float32),
                   jax.ShapeDtypeStruct((M, 1), jnp.float32)],
    )(x_p, W1p, deg_parts)


def _mm_layer(parts, h_cm, dinv, b_in, W, K3o, tm=512):
    _, K3i, M, _ = parts.shape
    K = K3i * DC
    _, Nw = W.shape

    def body(p0_ref, p1_ref, h_ref, d_ref, b_ref, w_ref, o_ref):
        agg = p0_ref[0] + p1_ref[0]
        dinv_t = d_ref[...]
        pre = jnp.maximum(
            (_from_cm(agg) + _from_cm(h_ref[...])) * dinv_t + b_ref[...],
            0.0)
        h2 = jnp.dot(pre, w_ref[...],
                     preferred_element_type=jnp.float32) * dinv_t
        o_ref[...] = _to_cm(h2, K3o, tm)

    return pl.pallas_call(
        body, grid=(M // tm,),
        in_specs=[pl.BlockSpec((1, K3i, tm, DC), lambda i: (0, 0, i, 0)),
                  pl.BlockSpec((1, K3i, tm, DC), lambda i: (1, 0, i, 0)),
                  pl.BlockSpec((K3i, tm, DC), lambda i: (0, i, 0)),
                  pl.BlockSpec((tm, 1), lambda i: (i, 0)),
                  pl.BlockSpec((1, K), lambda i: (0, 0)),
                  pl.BlockSpec((K, Nw), lambda i: (0, 0))],
        out_specs=pl.BlockSpec((K3o, tm, DC), lambda i: (0, i, 0)),
        out_shape=jax.ShapeDtypeStruct((K3o, M, DC), jnp.float32),
    )(parts, parts, h_cm, dinv, b_in.reshape(1, K), W)


def _post3(parts, h_cm, dinv, b_in, tm=512):
    _, K3i, M, _ = parts.shape
    K = K3i * DC

    def body(p0_ref, p1_ref, h_ref, d_ref, b_ref, o_ref):
        agg = p0_ref[0] + p1_ref[0]
        o_ref[...] = jnp.maximum(
            (_from_cm(agg) + _from_cm(h_ref[...])) * d_ref[...]
            + b_ref[...], 0.0)

    return pl.pallas_call(
        body, grid=(M // tm,),
        in_specs=[pl.BlockSpec((1, K3i, tm, DC), lambda i: (0, 0, i, 0)),
                  pl.BlockSpec((1, K3i, tm, DC), lambda i: (1, 0, i, 0)),
                  pl.BlockSpec((K3i, tm, DC), lambda i: (0, i, 0)),
                  pl.BlockSpec((tm, 1), lambda i: (i, 0)),
                  pl.BlockSpec((1, K), lambda i: (0, 0))],
        out_specs=pl.BlockSpec((tm, K), lambda i: (i, 0)),
        out_shape=jax.ShapeDtypeStruct((M, K), jnp.float32),
    )(parts, parts, h_cm, dinv, b_in.reshape(1, K))


def _mm(x, W, b_out=None, relu_out=False, tm=512):
    M, K = x.shape
    _, Nw = W.shape
    in_specs = [pl.BlockSpec((tm, K), lambda i: (i, 0)),
                pl.BlockSpec((K, Nw), lambda i: (0, 0))]
    args = [x, W]
    if b_out is not None:
        in_specs.append(pl.BlockSpec((1, Nw), lambda i: (0, 0)))
        args.append(b_out.reshape(1, Nw))

    def body(*refs):
        acc = jnp.dot(refs[0][...], refs[1][...],
                      preferred_element_type=jnp.float32)
        idx = 2
        if b_out is not None:
            acc = acc + refs[idx][...]; idx += 1
        if relu_out:
            acc = jnp.maximum(acc, 0.0)
        refs[idx][...] = acc

    return pl.pallas_call(
        body, grid=(M // tm,),
        in_specs=in_specs,
        out_specs=pl.BlockSpec((tm, Nw), lambda i: (i, 0)),
        out_shape=jax.ShapeDtypeStruct((M, Nw), jnp.float32),
    )(*args)


def _head_body(g_ref, ct_ref, wg1_ref, bg1_ref, wg2_ref, bg2_ref,
               wxt_ref, bxt_ref, wf1_ref, bf1_ref, wf2_ref, bf2_ref,
               wo_ref, bo_ref, o_ref):
    g = g_ref[...]
    g = jnp.maximum(jnp.dot(g, wg1_ref[...], preferred_element_type=jnp.float32)
                    + bg1_ref[...], 0.0)
    g = jnp.dot(g, wg2_ref[...], preferred_element_type=jnp.float32) + bg2_ref[...]
    xt = jnp.dot(ct_ref[...], wxt_ref[...], preferred_element_type=jnp.float32) \
        + bxt_ref[...]
    xc = jnp.concatenate([g, xt], axis=1)
    xc = jnp.maximum(jnp.dot(xc, wf1_ref[...], preferred_element_type=jnp.float32)
                     + bf1_ref[...], 0.0)
    xc = jnp.maximum(jnp.dot(xc, wf2_ref[...], preferred_element_type=jnp.float32)
                     + bf2_ref[...], 0.0)
    o_ref[...] = jnp.dot(xc, wo_ref[...], preferred_element_type=jnp.float32) \
        + bo_ref[...]


def _head(g, ct_flat, W_g1, b_g1, W_g2, b_g2, W_xt, b_xt,
          W_f1, b_f1, W_f2, b_f2, W_o, b_o):
    args = [g, ct_flat, W_g1, b_g1.reshape(1, -1), W_g2, b_g2.reshape(1, -1),
            W_xt, b_xt.reshape(1, -1), W_f1, b_f1.reshape(1, -1),
            W_f2, b_f2.reshape(1, -1), W_o, b_o.reshape(1, -1)]
    return pl.pallas_call(
        _head_body,
        out_shape=jax.ShapeDtypeStruct((B, 1), jnp.float32),
    )(*args)


def kernel(x, edge_index, batch, xo, W_c1, b_c1, W_c2, b_c2, W_c3, b_c3,
           W_g1, b_g1, W_g2, b_g2, cw1, cb1, cw2, cb2, cw3, cb3,
           W_xt, b_xt, W_f1, b_f1, W_f2, b_f2, W_o, b_o):
    src_p = jnp.pad(edge_index[0], (0, EP - E))
    dst_p = jnp.pad(edge_index[1], (0, EP - E), constant_values=N_PAD)
    dstr = dst_p.reshape(EROWS, 128)

    def srck(K3):
        off = (jnp.arange(K3, dtype=jnp.int32) * N_PAD)[:, None]
        return (src_p[None, :] + off).reshape(K3, EROWS, 128)

    deg_parts = _sc_degree(dst_p).reshape(2, N_PAD + 16)[:, :N_PAD]

    W1p = jnp.pad(W_c1, ((0, 2), (0, 2)))
    b1p = jnp.pad(b_c1, (0, 2))
    W2p = jnp.pad(W_c2, ((0, 2), (0, 4)))
    b2p = jnp.pad(b_c2, (0, 4))
    W3p = jnp.pad(W_c3, ((0, 4), (0, 8)))
    b3p = jnp.pad(b_c3, (0, 8))
    x_p = jnp.pad(x, ((0, N_PAD - N), (0, 2)))

    h1, dinv = _mm1_scaled(x_p, W1p, deg_parts, 5)
    agg1 = _sc_gcn_scatter(h1.reshape(5 * N_PAD, DC), srck(5), dstr, 5)
    h2 = _mm_layer(agg1.reshape(2, 5, N_PAD, DC), h1, dinv, b1p, W2p, 10)
    agg2 = _sc_gcn_scatter(h2.reshape(10 * N_PAD, DC), srck(10), dstr, 10)
    h3 = _mm_layer(agg2.reshape(2, 10, N_PAD, DC), h2, dinv, b2p, W3p, 20)
    agg3 = _sc_gcn_scatter(h3.reshape(20 * N_PAD, DC), srck(20), dstr, 20)
    h3f = _post3(agg3.reshape(2, 20, N_PAD, DC), h3, dinv, b3p)

    g = jax.ops.segment_max(h3f[:N, :312], batch, num_segments=B)

    def im2col(v, k):
        T = v.shape[1]
        cols = [v[:, i:T - k + 1 + i, :] for i in range(k)]
        return jnp.concatenate(cols, axis=2)

    def pool3(v):
        T3 = (v.shape[1] // 3) * 3
        return jnp.max(v[:, :T3].reshape(B, T3 // 3, 3, v.shape[2]), axis=2)

    v = xo.transpose(0, 2, 1)
    w1 = cw1.transpose(2, 1, 0).reshape(8, 32)
    c1 = im2col(v, 8).reshape(B * 713, 8)
    c1 = jnp.pad(c1, ((0, 91392 - B * 713), (0, 0)))
    y1 = _mm(c1, w1, b_out=cb1, relu_out=True, tm=448)[:B * 713]
    v = pool3(y1.reshape(B, 713, 32))

    w2 = cw2.transpose(2, 1, 0).reshape(8 * 32, 64)
    c2 = im2col(v, 8).reshape(B * 230, 256)
    c2 = jnp.pad(c2, ((0, 29696 - B * 230), (0, 0)))
    y2 = _mm(c2, w2, b_out=cb2, relu_out=True, tm=464)[:B * 230]
    v = pool3(y2.reshape(B, 230, 64))

    w3 = cw3.transpose(2, 1, 0).reshape(8 * 64, 128)
    c3 = im2col(v, 8).reshape(B * 69, 512)
    y3 = _mm(c3, w3, b_out=cb3, relu_out=True, tm=552)[:B * 69]
    v = pool3(y3.reshape(B, 69, 128))

    ct_flat = v.reshape(B, 23 * 128)
    W_xt_nhc = W_xt.reshape(128, 23, 128).transpose(1, 0, 2).reshape(2944, 128)

    return _head(g, ct_flat, W_g1, b_g1, W_g2, b_g2, W_xt_nhc, b_xt,
                 W_f1, b_f1, W_f2, b_f2, W_o, b_o)


# speedup vs baseline: 3.5264x; 1.1754x over previous
import functools

import jax
import jax.numpy as jnp
from jax import lax
from jax.experimental import pallas as pl
from jax.experimental.pallas import tpu as pltpu
from jax.experimental.pallas import tpu_sc as plsc

N = 50000
E = 800000
B = 128
L = 720

N_PAD = 51200
EP = 819200
GB = 400
DGB = 128
EROWS = EP // GB
TROWS = EROWS // 32
DC = 16
ZR = 200

_MESH = dict(core_axis_name="c", subcore_axis_name="s")


def _sc_degree(dst_p):
    per_tile = EP // 32
    nb = per_tile // DGB
    zchunk = N_PAD // 16

    @functools.partial(
        pl.kernel,
        out_type=jax.ShapeDtypeStruct((2 * (N_PAD + 16),), jnp.float32),
        mesh=plsc.VectorSubcoreMesh(**_MESH),
        scratch_types=[
            pltpu.VMEM((1, DGB), jnp.int32),
            pltpu.VMEM((DGB,), jnp.float32),
            pltpu.VMEM((8 * 16,), jnp.float32),
            pltpu.VMEM((zchunk,), jnp.float32),
            pltpu.VMEM_SHARED((N_PAD + 16,), jnp.float32),
        ],
    )
    def deg_kernel(dst_hbm, out_hbm, idxb, ones, zb, vbuf, acc):
        c = lax.axis_index("c")
        s = lax.axis_index("s")
        onev = jnp.ones((16,), jnp.float32)
        zerov = jnp.zeros((16,), jnp.float32)
        for t in range(8):
            ones[pl.ds(t * 16, 16)] = onev
            zb[pl.ds(t * 16, 16)] = zerov

        def zero_body(z, _):
            pltpu.sync_copy(zb, acc.at[pl.ds(s * zchunk + z * 128, 128)])
            return 0
        lax.fori_loop(0, zchunk // 128, zero_body, 0, unroll=False)
        plsc.subcore_barrier()

        base = (c * 16 + s) * per_tile

        def batch_body(j, _):
            pltpu.sync_copy(dst_hbm.at[pl.ds(base + j * DGB, DGB)],
                            idxb.at[0])
            pltpu.sync_copy(ones, acc.at[idxb.at[0]], add=True)
            return 0
        lax.fori_loop(0, nb, batch_body, 0, unroll=False)
        plsc.subcore_barrier()

        pltpu.sync_copy(acc.at[pl.ds(s * zchunk, zchunk)], vbuf)
        pltpu.sync_copy(vbuf,
                        out_hbm.at[pl.ds(c * (N_PAD + 16) + s * zchunk,
                                         zchunk)])

    return deg_kernel(dst_p)


def _sc_gcn_scatter(h2d, srck, dstr, K3):
    zpt = N_PAD // 16 // ZR

    @functools.partial(
        pl.kernel,
        out_type=jax.ShapeDtypeStruct((2 * K3 * N_PAD, DC), jnp.float32),
        mesh=plsc.VectorSubcoreMesh(**_MESH),
        compiler_params=pltpu.CompilerParams(use_tc_tiling_on_sc=False),
        scratch_types=[
            pltpu.VMEM((TROWS, GB), jnp.int32),
            pltpu.VMEM((TROWS, GB), jnp.int32),
            pltpu.VMEM((GB, DC), jnp.float32),
            pltpu.VMEM((GB, DC), jnp.float32),
            pltpu.VMEM((ZR, DC), jnp.float32),
            pltpu.VMEM((ZR, DC), jnp.float32),
            pltpu.VMEM_SHARED((N_PAD + 16, DC), jnp.float32),
            pltpu.SemaphoreType.DMA,
            pltpu.SemaphoreType.DMA,
        ],
    )
    def scatter_kernel(h_hbm, srck_hbm, dstr_hbm, out_hbm,
                       idxs, idxd, rb0, rb1, zbuf, bounce, acc, sem0, sem1):
        c = lax.axis_index("c")
        s = lax.axis_index("s")
        wid = c * 16 + s
        zerovf = jnp.zeros((16,), jnp.float32)
        for r in range(ZR):
            for q in range(DC // 16):
                zbuf[r, pl.ds(q * 16, 16)] = zerovf

        pltpu.sync_copy(dstr_hbm.at[pl.ds(wid * TROWS, TROWS)], idxd)

        for k in range(K3):
            pltpu.sync_copy(
                srck_hbm.at[k].at[pl.ds(wid * TROWS, TROWS)], idxs)

            def zero_body(z, _):
                pltpu.sync_copy(
                    zbuf, acc.at[pl.ds((s * zpt + z) * ZR, ZR)])
                return 0
            lax.fori_loop(0, zpt, zero_body, 0, unroll=False)
            plsc.subcore_barrier()

            pltpu.async_copy(h_hbm.at[idxs.at[0]], rb0, sem0)

            def pair_body(t, _):
                jj = 2 * t
                pltpu.make_async_copy(h_hbm.at[idxs.at[jj]], rb0,
                                      sem0).wait()
                pltpu.async_copy(h_hbm.at[idxs.at[jj + 1]], rb1, sem1)
                pltpu.sync_copy(rb0, acc.at[idxd.at[jj]], add=True)
                pltpu.make_async_copy(h_hbm.at[idxs.at[jj + 1]], rb1,
                                      sem1).wait()

                @pl.when(t < TROWS // 2 - 1)
                def _():
                    pltpu.async_copy(h_hbm.at[idxs.at[jj + 2]], rb0, sem0)
                pltpu.sync_copy(rb1, acc.at[idxd.at[jj + 1]], add=True)
                return 0
            lax.fori_loop(0, TROWS // 2, pair_body, 0, unroll=False)
            plsc.subcore_barrier()

            obase = (c * K3 + k) * N_PAD

            def wout_body(z, _):
                pltpu.sync_copy(acc.at[pl.ds((s * zpt + z) * ZR, ZR)],
                                bounce)
                pltpu.sync_copy(
                    bounce,
                    out_hbm.at[pl.ds(obase + (s * zpt + z) * ZR, ZR)])
                return 0
            lax.fori_loop(0, zpt, wout_body, 0, unroll=False)
            plsc.subcore_barrier()

    return scatter_kernel(h2d, srck, dstr)


def _to_cm(h, K3, tm):
    return h.reshape(tm, K3, DC).transpose(1, 0, 2)


def _from_cm(h_cm):
    K3, tm, _ = h_cm.shape
    return h_cm.transpose(1, 0, 2).reshape(tm, K3 * DC)


def _mm1_scaled(x_p, W1p, deg_parts, K3, tm=512):
    M, K = x_p.shape
    _, Nw = W1p.shape

    def body(x_ref, w_ref, dg_ref, o_ref, dinv_ref):
        dg = dg_ref[0, :] + dg_ref[1, :] + 1.0
        dinv = lax.rsqrt(dg)[:, None]
        h = jnp.dot(x_ref[...], w_ref[...],
                    preferred_element_type=jnp.float32) * dinv
        o_ref[...] = _to_cm(h, K3, tm)
        dinv_ref[...] = dinv

    return pl.pallas_call(
        body, grid=(M // tm,),
        in_specs=[pl.BlockSpec((tm, K), lambda i: (i, 0)),
                  pl.BlockSpec((K, Nw), lambda i: (0, 0)),
                  pl.BlockSpec((2, tm), lambda i: (0, i))],
        out_specs=[pl.BlockSpec((K3, tm, DC), lambda i: (0, i, 0)),
                   pl.BlockSpec((tm, 1), lambda i: (i, 0))],
        out_shape=[jax.ShapeDtypeStruct((K3, M, DC), jnp.float32),
                   jax.ShapeDtypeStruct((M, 1), jnp.float32)],
    )(x_p, W1p, deg_parts)


def _mm_layer(parts, h_cm, dinv, b_in, W, K3o, tm=512):
    _, K3i, M, _ = parts.shape
    K = K3i * DC
    _, Nw = W.shape

    def body(p0_ref, p1_ref, h_ref, d_ref, b_ref, w_ref, o_ref):
        agg = p0_ref[0] + p1_ref[0]
        dinv_t = d_ref[...]
        pre = jnp.maximum(
            (_from_cm(agg) + _from_cm(h_ref[...])) * dinv_t + b_ref[...],
            0.0)
        h2 = jnp.dot(pre, w_ref[...],
                     preferred_element_type=jnp.float32) * dinv_t
        o_ref[...] = _to_cm(h2, K3o, tm)

    return pl.pallas_call(
        body, grid=(M // tm,),
        in_specs=[pl.BlockSpec((1, K3i, tm, DC), lambda i: (0, 0, i, 0)),
                  pl.BlockSpec((1, K3i, tm, DC), lambda i: (1, 0, i, 0)),
                  pl.BlockSpec((K3i, tm, DC), lambda i: (0, i, 0)),
                  pl.BlockSpec((tm, 1), lambda i: (i, 0)),
                  pl.BlockSpec((1, K), lambda i: (0, 0)),
                  pl.BlockSpec((K, Nw), lambda i: (0, 0))],
        out_specs=pl.BlockSpec((K3o, tm, DC), lambda i: (0, i, 0)),
        out_shape=jax.ShapeDtypeStruct((K3o, M, DC), jnp.float32),
    )(parts, parts, h_cm, dinv, b_in.reshape(1, K), W)


def _post3(parts, h_cm, dinv, b_in, tm=512):
    _, K3i, M, _ = parts.shape
    K = K3i * DC

    def body(p0_ref, p1_ref, h_ref, d_ref, b_ref, o_ref):
        agg = p0_ref[0] + p1_ref[0]
        o_ref[...] = jnp.maximum(
            (_from_cm(agg) + _from_cm(h_ref[...])) * d_ref[...]
            + b_ref[...], 0.0)

    return pl.pallas_call(
        body, grid=(M // tm,),
        in_specs=[pl.BlockSpec((1, K3i, tm, DC), lambda i: (0, 0, i, 0)),
                  pl.BlockSpec((1, K3i, tm, DC), lambda i: (1, 0, i, 0)),
                  pl.BlockSpec((K3i, tm, DC), lambda i: (0, i, 0)),
                  pl.BlockSpec((tm, 1), lambda i: (i, 0)),
                  pl.BlockSpec((1, K), lambda i: (0, 0))],
        out_specs=pl.BlockSpec((tm, K), lambda i: (i, 0)),
        out_shape=jax.ShapeDtypeStruct((M, K), jnp.float32),
    )(parts, parts, h_cm, dinv, b_in.reshape(1, K))


def _mm(x, W, b_out=None, relu_out=False, tm=512):
    M, K = x.shape
    _, Nw = W.shape
    in_specs = [pl.BlockSpec((tm, K), lambda i: (i, 0)),
                pl.BlockSpec((K, Nw), lambda i: (0, 0))]
    args = [x, W]
    if b_out is not None:
        in_specs.append(pl.BlockSpec((1, Nw), lambda i: (0, 0)))
        args.append(b_out.reshape(1, Nw))

    def body(*refs):
        acc = jnp.dot(refs[0][...], refs[1][...],
                      preferred_element_type=jnp.float32)
        idx = 2
        if b_out is not None:
            acc = acc + refs[idx][...]; idx += 1
        if relu_out:
            acc = jnp.maximum(acc, 0.0)
        refs[idx][...] = acc

    return pl.pallas_call(
        body, grid=(M // tm,),
        in_specs=in_specs,
        out_specs=pl.BlockSpec((tm, Nw), lambda i: (i, 0)),
        out_shape=jax.ShapeDtypeStruct((M, Nw), jnp.float32),
    )(*args)


def _head_body(g_ref, ct_ref, wg1_ref, bg1_ref, wg2_ref, bg2_ref,
               wxt_ref, bxt_ref, wf1_ref, bf1_ref, wf2_ref, bf2_ref,
               wo_ref, bo_ref, o_ref):
    g = g_ref[...]
    g = jnp.maximum(jnp.dot(g, wg1_ref[...], preferred_element_type=jnp.float32)
                    + bg1_ref[...], 0.0)
    g = jnp.dot(g, wg2_ref[...], preferred_element_type=jnp.float32) + bg2_ref[...]
    xt = jnp.dot(ct_ref[...], wxt_ref[...], preferred_element_type=jnp.float32) \
        + bxt_ref[...]
    xc = jnp.concatenate([g, xt], axis=1)
    xc = jnp.maximum(jnp.dot(xc, wf1_ref[...], preferred_element_type=jnp.float32)
                     + bf1_ref[...], 0.0)
    xc = jnp.maximum(jnp.dot(xc, wf2_ref[...], preferred_element_type=jnp.float32)
                     + bf2_ref[...], 0.0)
    o_ref[...] = jnp.dot(xc, wo_ref[...], preferred_element_type=jnp.float32) \
        + bo_ref[...]


def _head(g, ct_flat, W_g1, b_g1, W_g2, b_g2, W_xt, b_xt,
          W_f1, b_f1, W_f2, b_f2, W_o, b_o):
    args = [g, ct_flat, W_g1, b_g1.reshape(1, -1), W_g2, b_g2.reshape(1, -1),
            W_xt, b_xt.reshape(1, -1), W_f1, b_f1.reshape(1, -1),
            W_f2, b_f2.reshape(1, -1), W_o, b_o.reshape(1, -1)]
    return pl.pallas_call(
        _head_body,
        out_shape=jax.ShapeDtypeStruct((B, 1), jnp.float32),
    )(*args)


def kernel(x, edge_index, batch, xo, W_c1, b_c1, W_c2, b_c2, W_c3, b_c3,
           W_g1, b_g1, W_g2, b_g2, cw1, cb1, cw2, cb2, cw3, cb3,
           W_xt, b_xt, W_f1, b_f1, W_f2, b_f2, W_o, b_o):
    src_p = jnp.pad(edge_index[0], (0, EP - E))
    dst_p = jnp.pad(edge_index[1], (0, EP - E), constant_values=N_PAD)
    dstr = dst_p.reshape(EROWS, GB)

    def srck(K3):
        off = (jnp.arange(K3, dtype=jnp.int32) * N_PAD)[:, None]
        return (src_p[None, :] + off).reshape(K3, EROWS, GB)

    deg_parts = _sc_degree(dst_p).reshape(2, N_PAD + 16)[:, :N_PAD]

    W1p = jnp.pad(W_c1, ((0, 2), (0, 2)))
    b1p = jnp.pad(b_c1, (0, 2))
    W2p = jnp.pad(W_c2, ((0, 2), (0, 4)))
    b2p = jnp.pad(b_c2, (0, 4))
    W3p = jnp.pad(W_c3, ((0, 4), (0, 8)))
    b3p = jnp.pad(b_c3, (0, 8))
    x_p = jnp.pad(x, ((0, N_PAD - N), (0, 2)))

    h1, dinv = _mm1_scaled(x_p, W1p, deg_parts, 5)
    agg1 = _sc_gcn_scatter(h1.reshape(5 * N_PAD, DC), srck(5), dstr, 5)
    h2 = _mm_layer(agg1.reshape(2, 5, N_PAD, DC), h1, dinv, b1p, W2p, 10)
    agg2 = _sc_gcn_scatter(h2.reshape(10 * N_PAD, DC), srck(10), dstr, 10)
    h3 = _mm_layer(agg2.reshape(2, 10, N_PAD, DC), h2, dinv, b2p, W3p, 20)
    agg3 = _sc_gcn_scatter(h3.reshape(20 * N_PAD, DC), srck(20), dstr, 20)
    h3f = _post3(agg3.reshape(2, 20, N_PAD, DC), h3, dinv, b3p)

    g = jax.ops.segment_max(h3f[:N, :312], batch, num_segments=B)

    def im2col(v, k):
        T = v.shape[1]
        cols = [v[:, i:T - k + 1 + i, :] for i in range(k)]
        return jnp.concatenate(cols, axis=2)

    def pool3(v):
        T3 = (v.shape[1] // 3) * 3
        return jnp.max(v[:, :T3].reshape(B, T3 // 3, 3, v.shape[2]), axis=2)

    v = xo.transpose(0, 2, 1)
    w1 = cw1.transpose(2, 1, 0).reshape(8, 32)
    c1 = im2col(v, 8).reshape(B * 713, 8)
    c1 = jnp.pad(c1, ((0, 91392 - B * 713), (0, 0)))
    y1 = _mm(c1, w1, b_out=cb1, relu_out=True, tm=448)[:B * 713]
    v = pool3(y1.reshape(B, 713, 32))

    w2 = cw2.transpose(2, 1, 0).reshape(8 * 32, 64)
    c2 = im2col(v, 8).reshape(B * 230, 256)
    c2 = jnp.pad(c2, ((0, 29696 - B * 230), (0, 0)))
    y2 = _mm(c2, w2, b_out=cb2, relu_out=True, tm=464)[:B * 230]
    v = pool3(y2.reshape(B, 230, 64))

    w3 = cw3.transpose(2, 1, 0).reshape(8 * 64, 128)
    c3 = im2col(v, 8).reshape(B * 69, 512)
    y3 = _mm(c3, w3, b_out=cb3, relu_out=True, tm=552)[:B * 69]
    v = pool3(y3.reshape(B, 69, 128))

    ct_flat = v.reshape(B, 23 * 128)
    W_xt_nhc = W_xt.reshape(128, 23, 128).transpose(1, 0, 2).reshape(2944, 128)

    return _head(g, ct_flat, W_g1, b_g1, W_g2, b_g2, W_xt_nhc, b_xt,
                 W_f1, b_f1, W_f2, b_f2, W_o, b_o)


# speedup vs baseline: 3.5430x; 1.0047x over previous
import functools

import jax
import jax.numpy as jnp
from jax import lax
from jax.experimental import pallas as pl
from jax.experimental.pallas import tpu as pltpu
from jax.experimental.pallas import tpu_sc as plsc

N = 50000
E = 800000
B = 128
L = 720

N_PAD = 51200
EP = 819200
GB = 400
DGB = 128
EROWS = EP // GB
TROWS = EROWS // 32
DC = 16
ZR = 400

_MESH = dict(core_axis_name="c", subcore_axis_name="s")


def _sc_degree(dst_p):
    per_tile = EP // 32
    nb = per_tile // DGB
    zchunk = N_PAD // 16

    @functools.partial(
        pl.kernel,
        out_type=jax.ShapeDtypeStruct((2 * (N_PAD + 16),), jnp.float32),
        mesh=plsc.VectorSubcoreMesh(**_MESH),
        scratch_types=[
            pltpu.VMEM((1, DGB), jnp.int32),
            pltpu.VMEM((DGB,), jnp.float32),
            pltpu.VMEM((8 * 16,), jnp.float32),
            pltpu.VMEM((zchunk,), jnp.float32),
            pltpu.VMEM_SHARED((N_PAD + 16,), jnp.float32),
        ],
    )
    def deg_kernel(dst_hbm, out_hbm, idxb, ones, zb, vbuf, acc):
        c = lax.axis_index("c")
        s = lax.axis_index("s")
        onev = jnp.ones((16,), jnp.float32)
        zerov = jnp.zeros((16,), jnp.float32)
        for t in range(8):
            ones[pl.ds(t * 16, 16)] = onev
            zb[pl.ds(t * 16, 16)] = zerov

        def zero_body(z, _):
            pltpu.sync_copy(zb, acc.at[pl.ds(s * zchunk + z * 128, 128)])
            return 0
        lax.fori_loop(0, zchunk // 128, zero_body, 0, unroll=False)
        plsc.subcore_barrier()

        base = (c * 16 + s) * per_tile

        def batch_body(j, _):
            pltpu.sync_copy(dst_hbm.at[pl.ds(base + j * DGB, DGB)],
                            idxb.at[0])
            pltpu.sync_copy(ones, acc.at[idxb.at[0]], add=True)
            return 0
        lax.fori_loop(0, nb, batch_body, 0, unroll=False)
        plsc.subcore_barrier()

        pltpu.sync_copy(acc.at[pl.ds(s * zchunk, zchunk)], vbuf)
        pltpu.sync_copy(vbuf,
                        out_hbm.at[pl.ds(c * (N_PAD + 16) + s * zchunk,
                                         zchunk)])

    return deg_kernel(dst_p)


def _sc_gcn_scatter(h2d, srck, dstr, K3):
    zpt = N_PAD // 16 // ZR

    @functools.partial(
        pl.kernel,
        out_type=jax.ShapeDtypeStruct((2 * K3 * N_PAD, DC), jnp.float32),
        mesh=plsc.VectorSubcoreMesh(**_MESH),
        compiler_params=pltpu.CompilerParams(use_tc_tiling_on_sc=False),
        scratch_types=[
            pltpu.VMEM((TROWS, GB), jnp.int32),
            pltpu.VMEM((TROWS, GB), jnp.int32),
            pltpu.VMEM((GB, DC), jnp.float32),
            pltpu.VMEM((GB, DC), jnp.float32),
            pltpu.VMEM((ZR, DC), jnp.float32),
            pltpu.VMEM((ZR, DC), jnp.float32),
            pltpu.VMEM_SHARED((N_PAD + 16, DC), jnp.float32),
            pltpu.SemaphoreType.DMA,
            pltpu.SemaphoreType.DMA,
        ],
    )
    def scatter_kernel(h_hbm, srck_hbm, dstr_hbm, out_hbm,
                       idxs, idxd, rb0, rb1, zbuf, bounce, acc, sem0, sem1):
        c = lax.axis_index("c")
        s = lax.axis_index("s")
        wid = c * 16 + s
        zerovf = jnp.zeros((16,), jnp.float32)
        for r in range(ZR):
            for q in range(DC // 16):
                zbuf[r, pl.ds(q * 16, 16)] = zerovf

        pltpu.sync_copy(dstr_hbm.at[pl.ds(wid * TROWS, TROWS)], idxd)

        for k in range(K3):
            pltpu.sync_copy(
                srck_hbm.at[k].at[pl.ds(wid * TROWS, TROWS)], idxs)

            def zero_body(z, _):
                pltpu.sync_copy(
                    zbuf, acc.at[pl.ds((s * zpt + z) * ZR, ZR)])
                return 0
            lax.fori_loop(0, zpt, zero_body, 0, unroll=False)
            plsc.subcore_barrier()

            pltpu.async_copy(h_hbm.at[idxs.at[0]], rb0, sem0)

            def pair_body(t, _):
                jj = 2 * t
                pltpu.make_async_copy(h_hbm.at[idxs.at[jj]], rb0,
                                      sem0).wait()
                pltpu.async_copy(h_hbm.at[idxs.at[jj + 1]], rb1, sem1)
                pltpu.sync_copy(rb0, acc.at[idxd.at[jj]], add=True)
                pltpu.make_async_copy(h_hbm.at[idxs.at[jj + 1]], rb1,
                                      sem1).wait()

                @pl.when(t < TROWS // 2 - 1)
                def _():
                    pltpu.async_copy(h_hbm.at[idxs.at[jj + 2]], rb0, sem0)
                pltpu.sync_copy(rb1, acc.at[idxd.at[jj + 1]], add=True)
                return 0
            lax.fori_loop(0, TROWS // 2, pair_body, 0, unroll=False)
            plsc.subcore_barrier()

            obase = (c * K3 + k) * N_PAD

            def wout_body(z, _):
                pltpu.sync_copy(acc.at[pl.ds((s * zpt + z) * ZR, ZR)],
                                bounce)
                pltpu.sync_copy(
                    bounce,
                    out_hbm.at[pl.ds(obase + (s * zpt + z) * ZR, ZR)])
                return 0
            lax.fori_loop(0, zpt, wout_body, 0, unroll=False)
            plsc.subcore_barrier()

    return scatter_kernel(h2d, srck, dstr)


def _to_cm(h, K3, tm):
    return h.reshape(tm, K3, DC).transpose(1, 0, 2)


def _from_cm(h_cm):
    K3, tm, _ = h_cm.shape
    return h_cm.transpose(1, 0, 2).reshape(tm, K3 * DC)


def _mm1_scaled(x_p, W1p, deg_parts, K3, tm=512):
    M, K = x_p.shape
    _, Nw = W1p.shape

    def body(x_ref, w_ref, dg_ref, o_ref, dinv_ref):
        dg = dg_ref[0, :] + dg_ref[1, :] + 1.0
        dinv = lax.rsqrt(dg)[:, None]
        h = jnp.dot(x_ref[...], w_ref[...],
                    preferred_element_type=jnp.float32) * dinv
        o_ref[...] = _to_cm(h, K3, tm)
        dinv_ref[...] = dinv

    return pl.pallas_call(
        body, grid=(M // tm,),
        in_specs=[pl.BlockSpec((tm, K), lambda i: (i, 0)),
                  pl.BlockSpec((K, Nw), lambda i: (0, 0)),
                  pl.BlockSpec((2, tm), lambda i: (0, i))],
        out_specs=[pl.BlockSpec((K3, tm, DC), lambda i: (0, i, 0)),
                   pl.BlockSpec((tm, 1), lambda i: (i, 0))],
        out_shape=[jax.ShapeDtypeStruct((K3, M, DC), jnp.float32),
                   jax.ShapeDtypeStruct((M, 1), jnp.float32)],
    )(x_p, W1p, deg_parts)


def _mm_layer(parts, h_cm, dinv, b_in, W, K3o, tm=512):
    _, K3i, M, _ = parts.shape
    K = K3i * DC
    _, Nw = W.shape

    def body(p0_ref, p1_ref, h_ref, d_ref, b_ref, w_ref, o_ref):
        agg = p0_ref[0] + p1_ref[0]
        dinv_t = d_ref[...]
        pre = jnp.maximum(
            (_from_cm(agg) + _from_cm(h_ref[...])) * dinv_t + b_ref[...],
            0.0)
        h2 = jnp.dot(pre, w_ref[...],
                     preferred_element_type=jnp.float32) * dinv_t
        o_ref[...] = _to_cm(h2, K3o, tm)

    return pl.pallas_call(
        body, grid=(M // tm,),
        in_specs=[pl.BlockSpec((1, K3i, tm, DC), lambda i: (0, 0, i, 0)),
                  pl.BlockSpec((1, K3i, tm, DC), lambda i: (1, 0, i, 0)),
                  pl.BlockSpec((K3i, tm, DC), lambda i: (0, i, 0)),
                  pl.BlockSpec((tm, 1), lambda i: (i, 0)),
                  pl.BlockSpec((1, K), lambda i: (0, 0)),
                  pl.BlockSpec((K, Nw), lambda i: (0, 0))],
        out_specs=pl.BlockSpec((K3o, tm, DC), lambda i: (0, i, 0)),
        out_shape=jax.ShapeDtypeStruct((K3o, M, DC), jnp.float32),
    )(parts, parts, h_cm, dinv, b_in.reshape(1, K), W)


def _post3(parts, h_cm, dinv, b_in, tm=512):
    _, K3i, M, _ = parts.shape
    K = K3i * DC

    def body(p0_ref, p1_ref, h_ref, d_ref, b_ref, o_ref):
        agg = p0_ref[0] + p1_ref[0]
        o_ref[...] = jnp.maximum(
            (_from_cm(agg) + _from_cm(h_ref[...])) * d_ref[...]
            + b_ref[...], 0.0)

    return pl.pallas_call(
        body, grid=(M // tm,),
        in_specs=[pl.BlockSpec((1, K3i, tm, DC), lambda i: (0, 0, i, 0)),
                  pl.BlockSpec((1, K3i, tm, DC), lambda i: (1, 0, i, 0)),
                  pl.BlockSpec((K3i, tm, DC), lambda i: (0, i, 0)),
                  pl.BlockSpec((tm, 1), lambda i: (i, 0)),
                  pl.BlockSpec((1, K), lambda i: (0, 0))],
        out_specs=pl.BlockSpec((tm, K), lambda i: (i, 0)),
        out_shape=jax.ShapeDtypeStruct((M, K), jnp.float32),
    )(parts, parts, h_cm, dinv, b_in.reshape(1, K))


def _mm(x, W, b_out=None, relu_out=False, tm=512):
    M, K = x.shape
    _, Nw = W.shape
    in_specs = [pl.BlockSpec((tm, K), lambda i: (i, 0)),
                pl.BlockSpec((K, Nw), lambda i: (0, 0))]
    args = [x, W]
    if b_out is not None:
        in_specs.append(pl.BlockSpec((1, Nw), lambda i: (0, 0)))
        args.append(b_out.reshape(1, Nw))

    def body(*refs):
        acc = jnp.dot(refs[0][...], refs[1][...],
                      preferred_element_type=jnp.float32)
        idx = 2
        if b_out is not None:
            acc = acc + refs[idx][...]; idx += 1
        if relu_out:
            acc = jnp.maximum(acc, 0.0)
        refs[idx][...] = acc

    return pl.pallas_call(
        body, grid=(M // tm,),
        in_specs=in_specs,
        out_specs=pl.BlockSpec((tm, Nw), lambda i: (i, 0)),
        out_shape=jax.ShapeDtypeStruct((M, Nw), jnp.float32),
    )(*args)


def _head_body(g_ref, ct_ref, wg1_ref, bg1_ref, wg2_ref, bg2_ref,
               wxt_ref, bxt_ref, wf1_ref, bf1_ref, wf2_ref, bf2_ref,
               wo_ref, bo_ref, o_ref):
    g = g_ref[...]
    g = jnp.maximum(jnp.dot(g, wg1_ref[...], preferred_element_type=jnp.float32)
                    + bg1_ref[...], 0.0)
    g = jnp.dot(g, wg2_ref[...], preferred_element_type=jnp.float32) + bg2_ref[...]
    xt = jnp.dot(ct_ref[...], wxt_ref[...], preferred_element_type=jnp.float32) \
        + bxt_ref[...]
    xc = jnp.concatenate([g, xt], axis=1)
    xc = jnp.maximum(jnp.dot(xc, wf1_ref[...], preferred_element_type=jnp.float32)
                     + bf1_ref[...], 0.0)
    xc = jnp.maximum(jnp.dot(xc, wf2_ref[...], preferred_element_type=jnp.float32)
                     + bf2_ref[...], 0.0)
    o_ref[...] = jnp.dot(xc, wo_ref[...], preferred_element_type=jnp.float32) \
        + bo_ref[...]


def _head(g, ct_flat, W_g1, b_g1, W_g2, b_g2, W_xt, b_xt,
          W_f1, b_f1, W_f2, b_f2, W_o, b_o):
    args = [g, ct_flat, W_g1, b_g1.reshape(1, -1), W_g2, b_g2.reshape(1, -1),
            W_xt, b_xt.reshape(1, -1), W_f1, b_f1.reshape(1, -1),
            W_f2, b_f2.reshape(1, -1), W_o, b_o.reshape(1, -1)]
    return pl.pallas_call(
        _head_body,
        out_shape=jax.ShapeDtypeStruct((B, 1), jnp.float32),
    )(*args)


def kernel(x, edge_index, batch, xo, W_c1, b_c1, W_c2, b_c2, W_c3, b_c3,
           W_g1, b_g1, W_g2, b_g2, cw1, cb1, cw2, cb2, cw3, cb3,
           W_xt, b_xt, W_f1, b_f1, W_f2, b_f2, W_o, b_o):
    src_p = jnp.pad(edge_index[0], (0, EP - E))
    dst_p = jnp.pad(edge_index[1], (0, EP - E), constant_values=N_PAD)
    dstr = dst_p.reshape(EROWS, GB)

    def srck(K3):
        off = (jnp.arange(K3, dtype=jnp.int32) * N_PAD)[:, None]
        return (src_p[None, :] + off).reshape(K3, EROWS, GB)

    deg_parts = _sc_degree(dst_p).reshape(2, N_PAD + 16)[:, :N_PAD]

    W1p = jnp.pad(W_c1, ((0, 2), (0, 2)))
    b1p = jnp.pad(b_c1, (0, 2))
    W2p = jnp.pad(W_c2, ((0, 2), (0, 4)))
    b2p = jnp.pad(b_c2, (0, 4))
    W3p = jnp.pad(W_c3, ((0, 4), (0, 8)))
    b3p = jnp.pad(b_c3, (0, 8))
    x_p = jnp.pad(x, ((0, N_PAD - N), (0, 2)))

    h1, dinv = _mm1_scaled(x_p, W1p, deg_parts, 5)
    agg1 = _sc_gcn_scatter(h1.reshape(5 * N_PAD, DC), srck(5), dstr, 5)
    h2 = _mm_layer(agg1.reshape(2, 5, N_PAD, DC), h1, dinv, b1p, W2p, 10)
    agg2 = _sc_gcn_scatter(h2.reshape(10 * N_PAD, DC), srck(10), dstr, 10)
    h3 = _mm_layer(agg2.reshape(2, 10, N_PAD, DC), h2, dinv, b2p, W3p, 20)
    agg3 = _sc_gcn_scatter(h3.reshape(20 * N_PAD, DC), srck(20), dstr, 20)
    h3f = _post3(agg3.reshape(2, 20, N_PAD, DC), h3, dinv, b3p)

    g = jax.ops.segment_max(h3f[:N, :312], batch, num_segments=B)

    def im2col(v, k):
        T = v.shape[1]
        cols = [v[:, i:T - k + 1 + i, :] for i in range(k)]
        return jnp.concatenate(cols, axis=2)

    def pool3(v):
        T3 = (v.shape[1] // 3) * 3
        return jnp.max(v[:, :T3].reshape(B, T3 // 3, 3, v.shape[2]), axis=2)

    v = xo.transpose(0, 2, 1)
    w1 = cw1.transpose(2, 1, 0).reshape(8, 32)
    c1 = im2col(v, 8).reshape(B * 713, 8)
    c1 = jnp.pad(c1, ((0, 91392 - B * 713), (0, 0)))
    y1 = _mm(c1, w1, b_out=cb1, relu_out=True, tm=448)[:B * 713]
    v = pool3(y1.reshape(B, 713, 32))

    w2 = cw2.transpose(2, 1, 0).reshape(8 * 32, 64)
    c2 = im2col(v, 8).reshape(B * 230, 256)
    c2 = jnp.pad(c2, ((0, 29696 - B * 230), (0, 0)))
    y2 = _mm(c2, w2, b_out=cb2, relu_out=True, tm=464)[:B * 230]
    v = pool3(y2.reshape(B, 230, 64))

    w3 = cw3.transpose(2, 1, 0).reshape(8 * 64, 128)
    c3 = im2col(v, 8).reshape(B * 69, 512)
    y3 = _mm(c3, w3, b_out=cb3, relu_out=True, tm=552)[:B * 69]
    v = pool3(y3.reshape(B, 69, 128))

    ct_flat = v.reshape(B, 23 * 128)
    W_xt_nhc = W_xt.reshape(128, 23, 128).transpose(1, 0, 2).reshape(2944, 128)

    return _head(g, ct_flat, W_g1, b_g1, W_g2, b_g2, W_xt_nhc, b_xt,
                 W_f1, b_f1, W_f2, b_f2, W_o, b_o)
